# baseline (speedup 1.0000x reference)
import jax
import jax.numpy as jnp
from jax import lax
from jax.experimental import pallas as pl
from jax.experimental.pallas import tpu as pltpu
from jax.experimental.pallas import tpu_sc as plsc

_C = 80
_H = 128
_W = 128
_K = 10
_BIG = 2**31 - 1


def _dense_kernel(off_ref, sz_ref, kp_ref, fpack_ref, ipack_ref, scores_ref):
    ninf = jnp.float32(-jnp.inf)
    row = jnp.full((1, _W), ninf, dtype=jnp.float32)
    colv = jnp.full((_H, 1), ninf, dtype=jnp.float32)

    pmax = jnp.zeros((_H, _W), dtype=jnp.float32)
    for c in range(_C):
        xc = kp_ref[c]
        up = jnp.concatenate([xc[1:], row], axis=0)
        dn = jnp.concatenate([row, xc[:-1]], axis=0)
        vy = jnp.maximum(xc, jnp.maximum(up, dn))
        lf = jnp.concatenate([vy[:, 1:], colv], axis=1)
        rt = jnp.concatenate([colv, vy[:, :-1]], axis=1)
        pooled = jnp.maximum(vy, jnp.maximum(lf, rt))
        sc_c = jnp.where(pooled == xc, xc, jnp.float32(0.0))
        scores_ref[c] = sc_c
        pmax = jnp.maximum(pmax, sc_c)

    hh = lax.broadcasted_iota(jnp.int32, (_H, _W), 0)
    ww = lax.broadcasted_iota(jnp.int32, (_H, _W), 1)
    pidx = hh * _W + ww
    big = jnp.int32(_BIG)

    wins = []
    for _ in range(_K):
        m = jnp.max(pmax)
        w = jnp.min(jnp.where(pmax == m, pidx, big))
        pmax = jnp.where(pidx == w, -1.0, pmax)
        wins.append(w)

    lane_w = lax.broadcasted_iota(jnp.int32, (_C, _W), 1)
    lane1 = lax.broadcasted_iota(jnp.int32, (1, _W), 1)

    fibs = []
    fidxs = []
    o0s, o1s, s0s, s1s = [], [], [], []
    for k in range(_K):
        w = wins[k]
        yi = w // _W
        xi = w - yi * _W

        slab = scores_ref[:, pl.ds(yi, 1), :].reshape(_C, _W)
        fib = jnp.sum(jnp.where(lane_w == xi, slab, 0.0), axis=1)
        fibs.append(fib)
        fidxs.append(w * _C + lax.iota(jnp.int32, _C))

        sel = lane1 == xi
        o0s.append(jnp.sum(jnp.where(sel, off_ref[0, pl.ds(yi, 1), :], 0.0)))
        o1s.append(jnp.sum(jnp.where(sel, off_ref[1, pl.ds(yi, 1), :], 0.0)))
        s0s.append(jnp.sum(jnp.where(sel, sz_ref[0, pl.ds(yi, 1), :], 0.0)))
        s1s.append(jnp.sum(jnp.where(sel, sz_ref[1, pl.ds(yi, 1), :], 0.0)))

    vals2d = jnp.stack(fibs)
    idx2d = jnp.stack(fidxs)
    cls_iota = lax.broadcasted_iota(jnp.int32, (_K, _C), 1)
    vcols, icols = [], []
    for _ in range(_K):
        m = jnp.max(vals2d, axis=1, keepdims=True)
        wc = jnp.min(jnp.where(vals2d == m, cls_iota, big),
                     axis=1, keepdims=True)
        hit = cls_iota == wc
        gi = jnp.sum(jnp.where(hit, idx2d, 0), axis=1, keepdims=True)
        vcols.append(m)
        icols.append(gi)
        vals2d = jnp.where(hit, jnp.float32(-1.0), vals2d)

    vals10 = jnp.concatenate(
        vcols + [jnp.full((_K, 16 - _K), -1.0, jnp.float32)], axis=1)
    idxs10 = jnp.concatenate(
        icols + [jnp.full((_K, 16 - _K), big, jnp.int32)], axis=1)

    pad6 = [jnp.float32(0.0)] * (16 - _K)
    dec = jnp.stack([
        jnp.stack(o0s + pad6), jnp.stack(o1s + pad6),
        jnp.stack(s0s + pad6), jnp.stack(s1s + pad6)])
    winv = jnp.concatenate(
        [jnp.stack(wins), jnp.full((16 - _K,), -1, jnp.int32)])

    fpack_ref[...] = jnp.concatenate(
        [vals10, dec, jnp.zeros((2, 16), jnp.float32)], axis=0)
    ipack_ref[...] = jnp.concatenate(
        [idxs10, winv[None], jnp.full((5, 16), big, jnp.int32)], axis=0)


def _sc_tail_kernel(f_hbm, i_hbm, packed_hbm, cls_hbm,
                    f_v, i_v, out_v, cls_v):
    cid = lax.axis_index("c")
    sid = lax.axis_index("s")

    @pl.when((cid == 0) & (sid == 0))
    def _():
        pltpu.sync_copy(f_hbm, f_v)
        pltpu.sync_copy(i_hbm, i_v)

        big = jnp.int32(_BIG)
        vals = [f_v[j] for j in range(_K)]
        idxs = [i_v[j] for j in range(_K)]
        winv = i_v[_K]
        do0 = f_v[_K]
        do1 = f_v[_K + 1]
        ds0 = f_v[_K + 2]
        ds1 = f_v[_K + 3]
        lane = lax.iota(jnp.int32, 16)

        b0v = jnp.zeros((16,), jnp.float32)
        b1v = jnp.zeros((16,), jnp.float32)
        b2v = jnp.zeros((16,), jnp.float32)
        b3v = jnp.zeros((16,), jnp.float32)
        scv = jnp.zeros((16,), jnp.float32)
        clv = jnp.zeros((16,), jnp.int32)

        for k in range(_K):
            vm = vals[0]
            for j in range(1, _K):
                vm = jnp.maximum(vm, vals[j])
            m = vm[0]
            for l in range(1, 16):
                m = jnp.maximum(m, vm[l])

            im = jnp.full((16,), big, jnp.int32)
            for j in range(_K):
                im = jnp.minimum(im, jnp.where(vals[j] == m, idxs[j], big))
            idx = im[0]
            for l in range(1, 16):
                idx = jnp.minimum(idx, im[l])
            for j in range(_K):
                vals[j] = jnp.where(idxs[j] == idx, -1.0, vals[j])

            sp = idx // _C
            cls = idx - sp * _C
            yi = sp // _W
            xi = sp - yi * _W
            y_f = yi.astype(jnp.float32)
            x_f = xi.astype(jnp.float32)

            o0 = jnp.float32(0.0)
            o1 = jnp.float32(0.0)
            s0 = jnp.float32(0.0)
            s1 = jnp.float32(0.0)
            for j in range(_K):
                hit = winv[j] == sp
                o0 = jnp.where(hit, do0[j], o0)
                o1 = jnp.where(hit, do1[j], o1)
                s0 = jnp.where(hit, ds0[j], s0)
                s1 = jnp.where(hit, ds1[j], s1)

            pos0 = y_f + o1
            pos1 = x_f + o0
            hw0 = s1 * 0.5
            hw1 = s0 * 0.5
            lim = jnp.float32(_W - 1)
            ksel = lane == k
            b0v = jnp.where(ksel, jnp.clip(pos0 - hw0, 0.0, lim) * 4.0, b0v)
            b1v = jnp.where(ksel, jnp.clip(pos1 - hw1, 0.0, lim) * 4.0, b1v)
            b2v = jnp.where(ksel, jnp.clip(pos0 + hw0, 0.0, lim) * 4.0, b2v)
            b3v = jnp.where(ksel, jnp.clip(pos1 + hw1, 0.0, lim) * 4.0, b3v)
            scv = jnp.where(ksel, m, scv)
            clv = jnp.where(ksel, cls, clv)

        out_v[pl.ds(0, 16)] = b0v
        out_v[pl.ds(16, 16)] = b1v
        out_v[pl.ds(32, 16)] = b2v
        out_v[pl.ds(48, 16)] = b3v
        out_v[pl.ds(64, 16)] = scv
        cls_v[...] = clv

        pltpu.sync_copy(out_v, packed_hbm)
        pltpu.sync_copy(cls_v, cls_hbm)


@jax.jit
def kernel(offset, size, keypoint):
    off = offset[0]
    sz = size[0]
    kp = keypoint[0]
    fpack, ipack = pl.pallas_call(
        _dense_kernel,
        out_shape=(
            jax.ShapeDtypeStruct((16, 16), jnp.float32),
            jax.ShapeDtypeStruct((16, 16), jnp.int32),
        ),
        scratch_shapes=[pltpu.VMEM((_C, _H, _W), jnp.float32)],
    )(off, sz, kp)

    mesh = plsc.VectorSubcoreMesh(core_axis_name="c", subcore_axis_name="s")
    sc_call = pl.kernel(
        _sc_tail_kernel,
        mesh=mesh,
        out_type=(
            jax.ShapeDtypeStruct((80,), jnp.float32),
            jax.ShapeDtypeStruct((16,), jnp.int32),
        ),
        scratch_types=[
            pltpu.VMEM((16, 16), jnp.float32),
            pltpu.VMEM((16, 16), jnp.int32),
            pltpu.VMEM((80,), jnp.float32),
            pltpu.VMEM((16,), jnp.int32),
        ],
    )
    packed, cls_p = sc_call(fpack, ipack)
    boxes = jnp.stack([packed[0:16][: _K], packed[16:32][: _K],
                       packed[32:48][: _K], packed[48:64][: _K]], axis=1)
    sc_scores = packed[64:80][: _K]
    cls = cls_p[: _K]
    return boxes, cls, sc_scores
